# Initial kernel scaffold; baseline (speedup 1.0000x reference)
import functools

import jax
import jax.numpy as jnp
from jax import lax
from jax.experimental import pallas as pl
from jax.experimental.pallas import tpu as pltpu
from jax.experimental.pallas import tpu_sc as plsc

N = 10000
D = 256
H = 128
EPS = 1e-5
NC = 2
NS = 16
CH = 128
NCHUNK = 80
T_EDGE = NCHUNK * CH
E_PAD = NS * T_EDGE
N_ACC = 10240
RPT = N // NS
DR = 125
BN = 1000

_mesh = plsc.VectorSubcoreMesh(
    core_axis_name="c", subcore_axis_name="s", num_cores=NC, num_subcores=NS)


@functools.partial(
    pl.kernel,
    out_type=jax.ShapeDtypeStruct((NC, N, 16), jnp.float32),
    mesh=_mesh,
    scratch_types=[
        pltpu.VMEM((NCHUNK, CH), jnp.int32),
        pltpu.VMEM((CH, 16), jnp.float32),
        pltpu.VMEM_SHARED((N_ACC, 16), jnp.float32),
    ],
)
def _sc_degree(dstp_hbm, out_hbm, dst_v, ones_v, acc_sh):
    c = lax.axis_index("c")
    s = lax.axis_index("s")
    pltpu.sync_copy(dstp_hbm.at[s], dst_v)
    for i in range(CH):
        ones_v[i, :] = jnp.full((16,), 1.0, jnp.float32)
    for k in range(N_ACC // NS // CH):
        row0 = s * (N_ACC // NS) + k * CH
        pltpu.sync_copy(ones_v, acc_sh.at[pl.ds(row0, CH)])
    plsc.subcore_barrier()

    def body(j, carry):
        jj = c * (NCHUNK // NC) + j
        pltpu.sync_copy(ones_v, acc_sh.at[dst_v.at[jj]], add=True)
        return carry

    lax.fori_loop(0, NCHUNK // NC, body, 0)
    plsc.subcore_barrier()
    for k in range(RPT // DR):
        row0 = s * RPT + k * DR
        pltpu.sync_copy(acc_sh.at[pl.ds(row0, DR)], ones_v.at[pl.ds(0, DR)])
        pltpu.sync_copy(ones_v.at[pl.ds(0, DR)], out_hbm.at[c].at[pl.ds(row0, DR)])


@functools.partial(
    pl.kernel,
    out_type=jax.ShapeDtypeStruct((NC, N, H), jnp.float32),
    mesh=_mesh,
    scratch_types=[
        pltpu.VMEM((NCHUNK, CH), jnp.int32),
        pltpu.VMEM((NCHUNK, CH), jnp.int32),
        pltpu.VMEM((CH, H), jnp.float32),
        pltpu.VMEM((CH, H), jnp.float32),
        pltpu.VMEM_SHARED((N_ACC, H), jnp.float32),
        pltpu.SemaphoreType.DMA,
        pltpu.SemaphoreType.DMA,
    ],
)
def _sc_gather_scatter(hs_hbm, srcp_hbm, dstp_hbm, out_hbm,
                       src_v, dst_v, buf_a, buf_b, acc_sh, sem_a, sem_b):
    c = lax.axis_index("c")
    s = lax.axis_index("s")
    table = hs_hbm.at[c]
    pltpu.sync_copy(srcp_hbm.at[s], src_v)
    pltpu.sync_copy(dstp_hbm.at[s], dst_v)
    for k in range(RPT // DR):
        row0 = s * RPT + k * DR
        pltpu.sync_copy(table.at[pl.ds(row0, DR)], buf_a.at[pl.ds(0, DR)])
        pltpu.sync_copy(buf_a.at[pl.ds(0, DR)], acc_sh.at[pl.ds(row0, DR)])
    plsc.subcore_barrier()

    pltpu.async_copy(table.at[src_v.at[0]], buf_a, sem_a)

    def body(i, carry):
        ja = 2 * i
        pltpu.async_copy(table.at[src_v.at[ja + 1]], buf_b, sem_b)
        pltpu.make_async_copy(table.at[src_v.at[ja]], buf_a, sem_a).wait()
        pltpu.sync_copy(buf_a, acc_sh.at[dst_v.at[ja]], add=True)

        @pl.when(i < NCHUNK // 2 - 1)
        def _():
            pltpu.async_copy(table.at[src_v.at[ja + 2]], buf_a, sem_a)

        pltpu.make_async_copy(table.at[src_v.at[ja + 1]], buf_b, sem_b).wait()
        pltpu.sync_copy(buf_b, acc_sh.at[dst_v.at[ja + 1]], add=True)
        return carry

    lax.fori_loop(0, NCHUNK // 2, body, 0)
    plsc.subcore_barrier()
    for k in range(RPT // DR):
        row0 = s * RPT + k * DR
        pltpu.sync_copy(acc_sh.at[pl.ds(row0, DR)], buf_a.at[pl.ds(0, DR)])
        pltpu.sync_copy(buf_a.at[pl.ds(0, DR)], out_hbm.at[c].at[pl.ds(row0, DR)])


def _dinv_of(deg_ref):
    d = deg_ref[0, :, 0:1] + deg_ref[1, :, 0:1] - 1.0
    return lax.rsqrt(d)


def _mm1_body(x_ref, w_ref, deg_ref, out_ref):
    dinv = _dinv_of(deg_ref)
    hp = jnp.dot(x_ref[...], w_ref[...], preferred_element_type=jnp.float32,
                 precision=lax.Precision.HIGHEST)
    hs = hp * dinv
    out_ref[0] = hs[:, :H]
    out_ref[1] = hs[:, H:]


def _mid_body(acc_ref, deg_ref, b_ref, g_ref, be_ref, w_ref, out_ref):
    dinv = _dinv_of(deg_ref)
    conv = jnp.concatenate([acc_ref[0], acc_ref[1]], axis=1) * dinv + b_ref[...]
    mu = jnp.mean(conv, axis=1, keepdims=True)
    var = jnp.mean((conv - mu) ** 2, axis=1, keepdims=True)
    h = jnp.maximum((conv - mu) * lax.rsqrt(var + EPS) * g_ref[...] + be_ref[...],
                    0.0)
    hs = jnp.dot(h, w_ref[...], preferred_element_type=jnp.float32,
                 precision=lax.Precision.HIGHEST) * dinv
    out_ref[0] = hs[:, :H]
    out_ref[1] = hs[:, H:]


def _fin_body(acc_ref, deg_ref, b_ref, g_ref, be_ref, out_ref):
    dinv = _dinv_of(deg_ref)
    conv = jnp.concatenate([acc_ref[0], acc_ref[1]], axis=1) * dinv + b_ref[...]
    mu = jnp.mean(conv, axis=1, keepdims=True)
    var = jnp.mean((conv - mu) ** 2, axis=1, keepdims=True)
    out_ref[...] = jnp.maximum(
        (conv - mu) * lax.rsqrt(var + EPS) * g_ref[...] + be_ref[...], 0.0)


_spec_deg = pl.BlockSpec((NC, BN, 16), lambda i: (0, i, 0))
_spec_acc = pl.BlockSpec((NC, BN, H), lambda i: (0, i, 0))
_spec_row = pl.BlockSpec((1, D), lambda i: (0, 0))
_spec_w = pl.BlockSpec((D, D), lambda i: (0, 0))

_mm1 = pl.pallas_call(
    _mm1_body,
    grid=(N // BN,),
    in_specs=[pl.BlockSpec((BN, D), lambda i: (i, 0)), _spec_w, _spec_deg],
    out_specs=_spec_acc,
    out_shape=jax.ShapeDtypeStruct((NC, N, H), jnp.float32),
)

_mid = pl.pallas_call(
    _mid_body,
    grid=(N // BN,),
    in_specs=[_spec_acc, _spec_deg, _spec_row, _spec_row, _spec_row, _spec_w],
    out_specs=_spec_acc,
    out_shape=jax.ShapeDtypeStruct((NC, N, H), jnp.float32),
)

_fin = pl.pallas_call(
    _fin_body,
    grid=(N // BN,),
    in_specs=[_spec_acc, _spec_deg, _spec_row, _spec_row, _spec_row],
    out_specs=pl.BlockSpec((BN, D), lambda i: (i, 0)),
    out_shape=jax.ShapeDtypeStruct((N, D), jnp.float32),
)


def kernel(x, edge_index, W1, b1, g1, be1, W2, b2, g2, be2, W3, b3, g3, be3):
    src = edge_index[0]
    dst = edge_index[1]
    pad = E_PAD - src.shape[0]
    srcp = jnp.concatenate([src, jnp.zeros((pad,), src.dtype)])
    dstp = jnp.concatenate([dst, jnp.full((pad,), N, dst.dtype)])
    srcp = srcp.reshape(NS, NCHUNK, CH)
    dstp = dstp.reshape(NS, NCHUNK, CH)

    degp = _sc_degree(dstp)

    b1r, g1r, be1r = b1.reshape(1, D), g1.reshape(1, D), be1.reshape(1, D)
    b2r, g2r, be2r = b2.reshape(1, D), g2.reshape(1, D), be2.reshape(1, D)
    b3r, g3r, be3r = b3.reshape(1, D), g3.reshape(1, D), be3.reshape(1, D)

    hs = _mm1(x, W1, degp)
    acc = _sc_gather_scatter(hs, srcp, dstp)
    hs = _mid(acc, degp, b1r, g1r, be1r, W2)
    acc = _sc_gather_scatter(hs, srcp, dstp)
    hs = _mid(acc, degp, b2r, g2r, be2r, W3)
    acc = _sc_gather_scatter(hs, srcp, dstp)
    return _fin(acc, degp, b3r, g3r, be3r)


# speedup vs baseline: 1.0655x; 1.0655x over previous
import functools

import jax
import jax.numpy as jnp
from jax import lax
from jax.experimental import pallas as pl
from jax.experimental.pallas import tpu as pltpu
from jax.experimental.pallas import tpu_sc as plsc

N = 10000
D = 256
H = 128
EPS = 1e-5
NC = 2
NS = 16
CH = 64
NCHUNK = 160
T_EDGE = NCHUNK * CH
E_PAD = NS * T_EDGE
N_ACC = 10240
RPT = N_ACC // NS
HALF = NCHUNK // 2
BN = 1000

_mesh = plsc.VectorSubcoreMesh(
    core_axis_name="c", subcore_axis_name="s", num_cores=NC, num_subcores=NS)


@functools.partial(
    pl.kernel,
    out_type=jax.ShapeDtypeStruct((NC, N_ACC, H), jnp.float32),
    mesh=_mesh,
    scratch_types=[
        pltpu.VMEM((HALF, CH), jnp.int32),
        pltpu.VMEM((HALF, CH), jnp.int32),
        pltpu.VMEM((CH, H), jnp.float32),
        pltpu.VMEM((CH, H), jnp.float32),
        pltpu.VMEM_SHARED((N_ACC, H), jnp.float32),
        pltpu.SemaphoreType.DMA,
        pltpu.SemaphoreType.DMA,
    ],
)
def _sc_gather_scatter(hs_hbm, srcp_hbm, dstp_hbm, zero_hbm, out_hbm,
                       src_v, dst_v, buf_a, buf_b, acc_sh, sem_a, sem_b):
    c = lax.axis_index("c")
    s = lax.axis_index("s")
    table = hs_hbm.at[c]
    pltpu.sync_copy(zero_hbm, buf_a)
    for k in range(RPT // CH):
        pltpu.sync_copy(buf_a, acc_sh.at[pl.ds(s * RPT + k * CH, CH)])
    plsc.subcore_barrier()

    for p in range(2):
        pltpu.sync_copy(srcp_hbm.at[s].at[pl.ds(p * HALF, HALF)], src_v)
        pltpu.sync_copy(dstp_hbm.at[s].at[pl.ds(p * HALF, HALF)], dst_v)
        pltpu.async_copy(table.at[src_v.at[0]], buf_a, sem_a)

        def body(i, carry):
            ja = 2 * i
            pltpu.async_copy(table.at[src_v.at[ja + 1]], buf_b, sem_b)
            pltpu.make_async_copy(table.at[src_v.at[ja]], buf_a, sem_a).wait()
            pltpu.sync_copy(buf_a, acc_sh.at[dst_v.at[ja]], add=True)

            @pl.when(i < HALF // 2 - 1)
            def _():
                pltpu.async_copy(table.at[src_v.at[ja + 2]], buf_a, sem_a)

            pltpu.make_async_copy(table.at[src_v.at[ja + 1]], buf_b, sem_b).wait()
            pltpu.sync_copy(buf_b, acc_sh.at[dst_v.at[ja + 1]], add=True)
            return carry

        lax.fori_loop(0, HALF // 2, body, 0)
    plsc.subcore_barrier()
    for k in range(RPT // CH):
        row0 = s * RPT + k * CH
        pltpu.sync_copy(acc_sh.at[pl.ds(row0, CH)], buf_a)
        pltpu.sync_copy(buf_a, out_hbm.at[c].at[pl.ds(row0, CH)])


def _dinv_of(deg_ref):
    d = deg_ref[0, :, 0:1] + 1.0
    return lax.rsqrt(d)


def _mm1_body(x_ref, w_ref, deg_ref, out_ref):
    dinv = _dinv_of(deg_ref)
    hp = jnp.dot(x_ref[...], w_ref[...], preferred_element_type=jnp.float32,
                 precision=lax.Precision.HIGHEST)
    hs = hp * dinv
    out_ref[0] = hs[:, :H]
    out_ref[1] = hs[:, H:]


def _norm_in(acc_ref, hs_ref, deg_ref, b_ref, g_ref, be_ref):
    dinv = _dinv_of(deg_ref)
    agg = (jnp.concatenate([acc_ref[0], acc_ref[1]], axis=1)
           + jnp.concatenate([hs_ref[0], hs_ref[1]], axis=1))
    conv = agg * dinv + b_ref[...]
    mu = jnp.mean(conv, axis=1, keepdims=True)
    var = jnp.mean((conv - mu) ** 2, axis=1, keepdims=True)
    h = jnp.maximum(
        (conv - mu) * lax.rsqrt(var + EPS) * g_ref[...] + be_ref[...], 0.0)
    return h, dinv


def _mid_body(acc_ref, hs_ref, deg_ref, b_ref, g_ref, be_ref, w_ref, out_ref):
    h, dinv = _norm_in(acc_ref, hs_ref, deg_ref, b_ref, g_ref, be_ref)
    hs = jnp.dot(h, w_ref[...], preferred_element_type=jnp.float32,
                 precision=lax.Precision.HIGHEST) * dinv
    out_ref[0] = hs[:, :H]
    out_ref[1] = hs[:, H:]


def _fin_body(acc_ref, hs_ref, deg_ref, b_ref, g_ref, be_ref, out_ref):
    h, _ = _norm_in(acc_ref, hs_ref, deg_ref, b_ref, g_ref, be_ref)
    out_ref[...] = h


_spec_deg = pl.BlockSpec((1, BN, H), lambda i: (0, i, 0))
_spec_acc = pl.BlockSpec((NC, BN, H), lambda i: (0, i, 0))
_spec_row = pl.BlockSpec((1, D), lambda i: (0, 0))
_spec_w = pl.BlockSpec((D, D), lambda i: (0, 0))

_mm1 = pl.pallas_call(
    _mm1_body,
    grid=(N // BN,),
    in_specs=[pl.BlockSpec((BN, D), lambda i: (i, 0)), _spec_w, _spec_deg],
    out_specs=_spec_acc,
    out_shape=jax.ShapeDtypeStruct((NC, N, H), jnp.float32),
)

_mid = pl.pallas_call(
    _mid_body,
    grid=(N // BN,),
    in_specs=[_spec_acc, _spec_acc, _spec_deg,
              _spec_row, _spec_row, _spec_row, _spec_w],
    out_specs=_spec_acc,
    out_shape=jax.ShapeDtypeStruct((NC, N, H), jnp.float32),
)

_fin = pl.pallas_call(
    _fin_body,
    grid=(N // BN,),
    in_specs=[_spec_acc, _spec_acc, _spec_deg, _spec_row, _spec_row, _spec_row],
    out_specs=pl.BlockSpec((BN, D), lambda i: (i, 0)),
    out_shape=jax.ShapeDtypeStruct((N, D), jnp.float32),
)


def kernel(x, edge_index, W1, b1, g1, be1, W2, b2, g2, be2, W3, b3, g3, be3):
    src = edge_index[0]
    dst = edge_index[1]
    pad = E_PAD - src.shape[0]
    srcp = jnp.concatenate([src, jnp.zeros((pad,), src.dtype)])
    dstp = jnp.concatenate([dst, jnp.full((pad,), N, dst.dtype)])
    srcp = srcp.reshape(NS, NCHUNK, CH)
    dstp = dstp.reshape(NS, NCHUNK, CH)
    zero = jnp.zeros((CH, H), jnp.float32)

    onest = jnp.ones((NC, 8, H), jnp.float32)
    degp = _sc_gather_scatter(onest, jnp.zeros_like(srcp), dstp, zero)

    b1r, g1r, be1r = b1.reshape(1, D), g1.reshape(1, D), be1.reshape(1, D)
    b2r, g2r, be2r = b2.reshape(1, D), g2.reshape(1, D), be2.reshape(1, D)
    b3r, g3r, be3r = b3.reshape(1, D), g3.reshape(1, D), be3.reshape(1, D)

    hs = _mm1(x, W1, degp)
    acc = _sc_gather_scatter(hs, srcp, dstp, zero)
    hs = _mid(acc, hs, degp, b1r, g1r, be1r, W2)
    acc = _sc_gather_scatter(hs, srcp, dstp, zero)
    hs = _mid(acc, hs, degp, b2r, g2r, be2r, W3)
    acc = _sc_gather_scatter(hs, srcp, dstp, zero)
    return _fin(acc, hs, degp, b3r, g3r, be3r)


# speedup vs baseline: 6.6116x; 6.2052x over previous
import functools

import jax
import jax.numpy as jnp
from jax import lax
from jax.experimental import pallas as pl
from jax.experimental.pallas import tpu as pltpu
from jax.experimental.pallas import tpu_sc as plsc

N = 10000
D = 256
H = 128
EPS = 1e-5
NC = 2
NS = 16
CH = 64
NCHUNK = 160
T_EDGE = NCHUNK * CH
E_PAD = NS * T_EDGE
N_ACC = 10240
RPT = N_ACC // NS
HALF = NCHUNK // 2
BN = 1000

_mesh = plsc.VectorSubcoreMesh(
    core_axis_name="c", subcore_axis_name="s", num_cores=NC, num_subcores=NS)


@functools.partial(
    pl.kernel,
    out_type=jax.ShapeDtypeStruct((NC, N_ACC, H), jnp.float32),
    mesh=_mesh,
    scratch_types=[
        pltpu.VMEM((HALF, CH), jnp.int32),
        pltpu.VMEM((HALF, CH), jnp.int32),
        pltpu.VMEM((CH, H), jnp.float32),
        pltpu.VMEM((CH, H), jnp.float32),
        pltpu.VMEM_SHARED((N_ACC, H), jnp.float32),
        pltpu.SemaphoreType.DMA,
        pltpu.SemaphoreType.DMA,
    ],
)
def _sc_gather_scatter(hs_hbm, srcp_hbm, dstp_hbm, zero_hbm, out_hbm,
                       src_v, dst_v, buf_a, buf_b, acc_sh, sem_a, sem_b):
    c = lax.axis_index("c")
    s = lax.axis_index("s")
    table = hs_hbm.at[c]
    pltpu.sync_copy(zero_hbm, buf_a)
    for k in range(RPT // CH):
        pltpu.sync_copy(buf_a, acc_sh.at[pl.ds(s * RPT + k * CH, CH)])
    plsc.subcore_barrier()

    for p in range(2):
        pltpu.sync_copy(srcp_hbm.at[s].at[pl.ds(p * HALF, HALF)], src_v)
        pltpu.sync_copy(dstp_hbm.at[s].at[pl.ds(p * HALF, HALF)], dst_v)
        pltpu.async_copy(table.at[src_v.at[0]], buf_a, sem_a)

        def body(i, carry):
            ja = 2 * i
            pltpu.async_copy(table.at[src_v.at[ja + 1]], buf_b, sem_b)
            pltpu.make_async_copy(table.at[src_v.at[ja]], buf_a, sem_a).wait()
            pltpu.sync_copy(buf_a, acc_sh.at[dst_v.at[ja]], add=True)

            @pl.when(i < HALF // 2 - 1)
            def _():
                pltpu.async_copy(table.at[src_v.at[ja + 2]], buf_a, sem_a)

            pltpu.make_async_copy(table.at[src_v.at[ja + 1]], buf_b, sem_b).wait()
            pltpu.sync_copy(buf_b, acc_sh.at[dst_v.at[ja + 1]], add=True)
            return carry

        lax.fori_loop(0, HALF // 2, body, 0)
    plsc.subcore_barrier()
    for k in range(RPT // CH):
        row0 = s * RPT + k * CH
        pltpu.sync_copy(acc_sh.at[pl.ds(row0, CH)], buf_a)
        pltpu.sync_copy(buf_a, out_hbm.at[c].at[pl.ds(row0, CH)])


def _dinv_of(deg_ref):
    d = deg_ref[0, :, 0:1] + 1.0
    return lax.rsqrt(d)


def _mm1_body(x_ref, w_ref, deg_ref, out_ref):
    dinv = _dinv_of(deg_ref)
    hp = jnp.dot(x_ref[...], w_ref[...], preferred_element_type=jnp.float32,
                 precision=lax.Precision.HIGHEST)
    hs = hp * dinv
    out_ref[0] = hs[:, :H]
    out_ref[1] = hs[:, H:]


def _norm_in(acc_ref, hs_ref, deg_ref, b_ref, g_ref, be_ref):
    dinv = _dinv_of(deg_ref)
    agg = (jnp.concatenate([acc_ref[0], acc_ref[1]], axis=1)
           + jnp.concatenate([hs_ref[0], hs_ref[1]], axis=1))
    conv = agg * dinv + b_ref[...]
    mu = jnp.mean(conv, axis=1, keepdims=True)
    var = jnp.mean((conv - mu) ** 2, axis=1, keepdims=True)
    h = jnp.maximum(
        (conv - mu) * lax.rsqrt(var + EPS) * g_ref[...] + be_ref[...], 0.0)
    return h, dinv


def _mid_body(acc_ref, hs_ref, deg_ref, b_ref, g_ref, be_ref, w_ref, out_ref):
    h, dinv = _norm_in(acc_ref, hs_ref, deg_ref, b_ref, g_ref, be_ref)
    hs = jnp.dot(h, w_ref[...], preferred_element_type=jnp.float32,
                 precision=lax.Precision.HIGHEST) * dinv
    out_ref[0] = hs[:, :H]
    out_ref[1] = hs[:, H:]


def _fin_body(acc_ref, hs_ref, deg_ref, b_ref, g_ref, be_ref, out_ref):
    h, _ = _norm_in(acc_ref, hs_ref, deg_ref, b_ref, g_ref, be_ref)
    out_ref[...] = h


_spec_deg = pl.BlockSpec((1, BN, H), lambda i: (0, i, 0))
_spec_acc = pl.BlockSpec((NC, BN, H), lambda i: (0, i, 0))
_spec_row = pl.BlockSpec((1, D), lambda i: (0, 0))
_spec_w = pl.BlockSpec((D, D), lambda i: (0, 0))

_mm1 = pl.pallas_call(
    _mm1_body,
    grid=(N // BN,),
    in_specs=[pl.BlockSpec((BN, D), lambda i: (i, 0)), _spec_w, _spec_deg],
    out_specs=_spec_acc,
    out_shape=jax.ShapeDtypeStruct((NC, N, H), jnp.float32),
)

_mid = pl.pallas_call(
    _mid_body,
    grid=(N // BN,),
    in_specs=[_spec_acc, _spec_acc, _spec_deg,
              _spec_row, _spec_row, _spec_row, _spec_w],
    out_specs=_spec_acc,
    out_shape=jax.ShapeDtypeStruct((NC, N, H), jnp.float32),
)

_fin = pl.pallas_call(
    _fin_body,
    grid=(N // BN,),
    in_specs=[_spec_acc, _spec_acc, _spec_deg, _spec_row, _spec_row, _spec_row],
    out_specs=pl.BlockSpec((BN, D), lambda i: (i, 0)),
    out_shape=jax.ShapeDtypeStruct((N, D), jnp.float32),
)


def kernel(x, edge_index, W1, b1, g1, be1, W2, b2, g2, be2, W3, b3, g3, be3):
    src = edge_index[0]
    dst = edge_index[1]
    pad = E_PAD - src.shape[0]
    srcp = jnp.concatenate([src, jnp.zeros((pad,), src.dtype)])
    dstp = jnp.concatenate([dst, jnp.full((pad,), N, dst.dtype)])
    srcp = srcp.reshape(NS, NCHUNK, CH)
    dstp = dstp.reshape(NS, NCHUNK, CH)
    zero = jnp.zeros((CH, H), jnp.float32)

    deg_rows = 2048
    onest = jnp.ones((NC, deg_rows, H), jnp.float32)
    srcd = (jnp.arange(E_PAD, dtype=srcp.dtype) % deg_rows).reshape(
        NS, NCHUNK, CH)
    degp = _sc_gather_scatter(onest, srcd, dstp, zero)

    b1r, g1r, be1r = b1.reshape(1, D), g1.reshape(1, D), be1.reshape(1, D)
    b2r, g2r, be2r = b2.reshape(1, D), g2.reshape(1, D), be2.reshape(1, D)
    b3r, g3r, be3r = b3.reshape(1, D), g3.reshape(1, D), be3.reshape(1, D)

    hs = _mm1(x, W1, degp)
    acc = _sc_gather_scatter(hs, srcp, dstp, zero)
    hs = _mid(acc, hs, degp, b1r, g1r, be1r, W2)
    acc = _sc_gather_scatter(hs, srcp, dstp, zero)
    hs = _mid(acc, hs, degp, b2r, g2r, be2r, W3)
    acc = _sc_gather_scatter(hs, srcp, dstp, zero)
    return _fin(acc, hs, degp, b3r, g3r, be3r)
